# baseline (speedup 1.0000x reference)
import functools

import jax
import jax.numpy as jnp
from jax import lax
from jax.experimental import pallas as pl
from jax.experimental.pallas import tpu as pltpu
from jax.experimental.pallas import tpu_sc as plsc

N = 2048
D = 768
H = 768
E = 8
K = 2
T = 256
NT = 24
S = NT * T
NW = 32
SLOTS_W = S // NW
TOKS_W = (N * K) // NW


def _route_body(logits_ref, probs_ref, mask_ref, dest_ref, wflat_ref,
                meta_ref):
    l = logits_ref[...]
    m = jnp.max(l, axis=-1, keepdims=True)
    ex = jnp.exp(l - m)
    probs = ex / jnp.sum(ex, axis=-1, keepdims=True)
    probs_ref[...] = probs

    iota = lax.broadcasted_iota(jnp.int32, (N, E), 1)
    m1 = jnp.max(probs, axis=-1, keepdims=True)
    i1 = jnp.min(jnp.where(probs == m1, iota, E), axis=-1, keepdims=True)
    oh1 = iota == i1
    probs2 = jnp.where(oh1, -1.0, probs)
    m2 = jnp.max(probs2, axis=-1, keepdims=True)
    i2 = jnp.min(jnp.where(probs2 == m2, iota, E), axis=-1, keepdims=True)
    oh2 = iota == i2
    denom = m1 + m2 + 1e-8
    w1n = m1 / denom
    w2n = m2 / denom
    mask_ref[...] = jnp.where(oh1, w1n, 0.0) + jnp.where(oh2, w2n, 0.0)

    ohf = (oh1 | oh2).astype(jnp.float32)
    s = ohf
    d = 1
    while d < N:
        s = jnp.concatenate([jnp.zeros((d, E), jnp.float32), s[:-d]], axis=0) + s
        d *= 2
    excl = s - ohf
    counts = s[N - 1:N, :]
    counts_i = counts.astype(jnp.int32)
    ptiles = (counts_i + (T - 1)) >> 8
    padded = (ptiles << 8).astype(jnp.float32)

    padded_b = jnp.broadcast_to(padded, (N, E))
    poff1 = jnp.sum(jnp.where(iota < i1, padded_b, 0.0), axis=1, keepdims=True)
    poff2 = jnp.sum(jnp.where(iota < i2, padded_b, 0.0), axis=1, keepdims=True)
    rank1 = jnp.sum(jnp.where(iota == i1, excl, 0.0), axis=1, keepdims=True)
    rank2 = jnp.sum(jnp.where(iota == i2, excl, 0.0), axis=1, keepdims=True)
    dest1 = (poff1 + rank1).astype(jnp.int32)
    dest2 = (poff2 + rank2).astype(jnp.int32)
    kio = lax.broadcasted_iota(jnp.int32, (N, K), 1)
    dest_ref[...] = jnp.where(kio == 0, jnp.broadcast_to(dest1, (N, K)),
                              jnp.broadcast_to(dest2, (N, K)))
    wflat_ref[...] = jnp.where(kio == 0, jnp.broadcast_to(w1n, (N, K)),
                               jnp.broadcast_to(w2n, (N, K)))

    utri = (lax.broadcasted_iota(jnp.int32, (E, E), 0)
            <= lax.broadcasted_iota(jnp.int32, (E, E), 1)).astype(jnp.float32)
    pend = jax.lax.dot_general(padded, utri, (((1,), (0,)), ((), ())),
                               precision=jax.lax.Precision.HIGHEST)
    pend_b = jnp.broadcast_to(pend, (32, E))
    it = (lax.broadcasted_iota(jnp.int32, (32, E), 0) * T).astype(jnp.float32)
    te = jnp.sum((pend_b <= it).astype(jnp.int32), axis=1, keepdims=True)
    n_tiles = jnp.sum(ptiles, axis=1, keepdims=True)
    rio = lax.broadcasted_iota(jnp.int32, (32, 1), 0)
    meta_ref[...] = jnp.where(rio < NT, te, jnp.broadcast_to(n_tiles, (32, 1)))


def _route_call(gate_logits):
    return pl.pallas_call(
        _route_body,
        grid=(1,),
        in_specs=[pl.BlockSpec((N, E), lambda i: (0, 0))],
        out_specs=[
            pl.BlockSpec((N, E), lambda i: (0, 0)),
            pl.BlockSpec((N, E), lambda i: (0, 0)),
            pl.BlockSpec((N, K), lambda i: (0, 0)),
            pl.BlockSpec((N, K), lambda i: (0, 0)),
            pl.BlockSpec((32, 1), lambda i: (0, 0)),
        ],
        out_shape=[
            jax.ShapeDtypeStruct((N, E), jnp.float32),
            jax.ShapeDtypeStruct((N, E), jnp.float32),
            jax.ShapeDtypeStruct((N, K), jnp.int32),
            jax.ShapeDtypeStruct((N, K), jnp.float32),
            jax.ShapeDtypeStruct((32, 1), jnp.int32),
        ],
    )(gate_logits)


def _sc_sort_gather(dest_flat, x):
    mesh = plsc.VectorSubcoreMesh(core_axis_name="c", subcore_axis_name="s")

    @functools.partial(
        pl.kernel, mesh=mesh,
        out_type=jax.ShapeDtypeStruct((S, D), jnp.float32),
        compiler_params=pltpu.CompilerParams(needs_layout_passes=False),
        scratch_types=[
            pltpu.VMEM((N * K,), jnp.int32),
            pltpu.VMEM((256,), jnp.int32),
            pltpu.VMEM((SLOTS_W // 2, D), jnp.float32),
            pltpu.SemaphoreType.DMA,
        ],
    )
    def body(dest_hbm, x_hbm, xs_hbm, dest_v, tok_v, rows_v, sem):
        wid = lax.axis_index("s") * 2 + lax.axis_index("c")
        lo = wid * SLOTS_W
        pltpu.sync_copy(dest_hbm, dest_v)
        for j in range(256 // 16):
            tok_v[pl.ds(j * 16, 16)] = jnp.zeros((16,), jnp.int32)

        def step(c, carry):
            d16 = dest_v[pl.ds(c * 16, 16)]
            p16 = lax.iota(jnp.int32, 16) + c * 16
            t16 = p16 >> 1
            msk = (d16 >= lo) & (d16 < lo + SLOTS_W)
            plsc.store_scatter(tok_v, [d16 - lo], t16, mask=msk)
            return carry

        lax.fori_loop(0, (N * K) // 16, step, 0)
        for hchunk in range(2):
            idx = tok_v.at[pl.ds(hchunk * (SLOTS_W // 2), SLOTS_W // 2)]
            pltpu.async_copy(x_hbm.at[idx], rows_v, sem).wait()
            pltpu.sync_copy(
                rows_v,
                xs_hbm.at[pl.ds(lo + hchunk * (SLOTS_W // 2), SLOTS_W // 2)])

    return body(dest_flat, x)


def _gmm_body(m_ref, xs_ref, W1_ref, b1_ref, W2_ref, b2_ref, ys_ref):
    @pl.when(pl.program_id(0) < m_ref[NT])
    def _():
        xb = xs_ref[...].astype(jnp.bfloat16)
        h = jnp.dot(xb, W1_ref[0], preferred_element_type=jnp.float32)
        h = jnp.maximum(h + b1_ref[0], 0.0).astype(jnp.bfloat16)
        y = jnp.dot(h, W2_ref[0], preferred_element_type=jnp.float32)
        ys_ref[...] = y + b2_ref[0]


def _gmm_call(meta_flat, xs, W1b, b1r, W2b, b2r):
    def wmap(i, m):
        e = jnp.minimum(m[i], E - 1)
        return (e, 0, 0)

    grid_spec = pltpu.PrefetchScalarGridSpec(
        num_scalar_prefetch=1,
        grid=(NT,),
        in_specs=[
            pl.BlockSpec((T, D), lambda i, m: (i, 0)),
            pl.BlockSpec((1, D, H), wmap),
            pl.BlockSpec((1, 1, H), wmap),
            pl.BlockSpec((1, H, H), wmap),
            pl.BlockSpec((1, 1, H), wmap),
        ],
        out_specs=pl.BlockSpec((T, H), lambda i, m: (i, 0)),
    )
    return pl.pallas_call(
        _gmm_body,
        grid_spec=grid_spec,
        out_shape=jax.ShapeDtypeStruct((S, H), jnp.float32),
    )(meta_flat, xs, W1b, b1r, W2b, b2r)


def _sc_pair_gather(dest_flat, ys):
    mesh = plsc.VectorSubcoreMesh(core_axis_name="c", subcore_axis_name="s")

    @functools.partial(
        pl.kernel, mesh=mesh,
        out_type=jax.ShapeDtypeStruct((N * K, H), jnp.float32),
        scratch_types=[
            pltpu.VMEM((TOKS_W,), jnp.int32),
            pltpu.VMEM((TOKS_W, H), jnp.float32),
            pltpu.SemaphoreType.DMA,
        ],
    )
    def body(dest_hbm, y_hbm, yp_hbm, idx_v, rows_v, sem):
        wid = lax.axis_index("s") * 2 + lax.axis_index("c")
        base = wid * TOKS_W
        pltpu.sync_copy(dest_hbm.at[pl.ds(base, TOKS_W)], idx_v)
        pltpu.async_copy(y_hbm.at[idx_v], rows_v, sem).wait()
        pltpu.sync_copy(rows_v, yp_hbm.at[pl.ds(base, TOKS_W)])

    return body(dest_flat, ys)


TN = 256


def _combine_body(yp_ref, w_ref, out_ref):
    yp = yp_ref[...]
    w = w_ref[...]
    kio = lax.broadcasted_iota(jnp.int32, (TN, K), 1)
    w0 = jnp.sum(jnp.where(kio == 0, w, 0.0), axis=1, keepdims=True)
    w1 = jnp.sum(jnp.where(kio == 1, w, 0.0), axis=1, keepdims=True)
    out_ref[...] = yp[:, 0, :] * w0 + yp[:, 1, :] * w1


def _combine_call(yp, wflat):
    return pl.pallas_call(
        _combine_body,
        grid=(N // TN,),
        in_specs=[
            pl.BlockSpec((TN, K, H), lambda i: (i, 0, 0)),
            pl.BlockSpec((TN, K), lambda i: (i, 0)),
        ],
        out_specs=pl.BlockSpec((TN, H), lambda i: (i, 0)),
        out_shape=jax.ShapeDtypeStruct((N, H), jnp.float32),
    )(yp, wflat)


def kernel(x, Wg, bg, W1, b1, W2, b2):
    gate_logits = x @ Wg + bg
    probs, mask, dest, wflat, meta = _route_call(gate_logits)
    dest_flat = dest.reshape(N * K)
    meta_flat = meta.reshape(32)
    W1b = W1.astype(jnp.bfloat16)
    W2b = W2.astype(jnp.bfloat16)
    b1r = b1[:, None, :]
    b2r = b2[:, None, :]
    xs = _sc_sort_gather(dest_flat, x)
    ys = _gmm_call(meta_flat, xs, W1b, b1r, W2b, b2r)
    yp = _sc_pair_gather(dest_flat, ys).reshape(N, K, H)
    out = _combine_call(yp, wflat)
    return (out, probs, mask)


# speedup vs baseline: 1.0107x; 1.0107x over previous
import functools

import jax
import jax.numpy as jnp
from jax import lax
from jax.experimental import pallas as pl
from jax.experimental.pallas import tpu as pltpu
from jax.experimental.pallas import tpu_sc as plsc

N = 2048
D = 768
H = 768
E = 8
K = 2
T = 256
NT = 24
S = NT * T
NW = 32
SLOTS_W = S // NW
TOKS_W = (N * K) // NW


def _route_body(logits_ref, probs_ref, mask_ref, dest_ref, wflat_ref,
                meta_ref):
    l = logits_ref[...]
    m = jnp.max(l, axis=-1, keepdims=True)
    ex = jnp.exp(l - m)
    probs = ex / jnp.sum(ex, axis=-1, keepdims=True)
    probs_ref[...] = probs

    iota = lax.broadcasted_iota(jnp.int32, (N, E), 1)
    m1 = jnp.max(probs, axis=-1, keepdims=True)
    i1 = jnp.min(jnp.where(probs == m1, iota, E), axis=-1, keepdims=True)
    oh1 = iota == i1
    probs2 = jnp.where(oh1, -1.0, probs)
    m2 = jnp.max(probs2, axis=-1, keepdims=True)
    i2 = jnp.min(jnp.where(probs2 == m2, iota, E), axis=-1, keepdims=True)
    oh2 = iota == i2
    denom = m1 + m2 + 1e-8
    w1n = m1 / denom
    w2n = m2 / denom
    mask_ref[...] = jnp.where(oh1, w1n, 0.0) + jnp.where(oh2, w2n, 0.0)

    ohf = (oh1 | oh2).astype(jnp.float32)
    s = ohf
    d = 1
    while d < N:
        s = jnp.concatenate([jnp.zeros((d, E), jnp.float32), s[:-d]], axis=0) + s
        d *= 2
    excl = s - ohf
    counts = s[N - 1:N, :]
    counts_i = counts.astype(jnp.int32)
    ptiles = (counts_i + (T - 1)) >> 8
    padded = (ptiles << 8).astype(jnp.float32)

    padded_b = jnp.broadcast_to(padded, (N, E))
    poff1 = jnp.sum(jnp.where(iota < i1, padded_b, 0.0), axis=1, keepdims=True)
    poff2 = jnp.sum(jnp.where(iota < i2, padded_b, 0.0), axis=1, keepdims=True)
    rank1 = jnp.sum(jnp.where(iota == i1, excl, 0.0), axis=1, keepdims=True)
    rank2 = jnp.sum(jnp.where(iota == i2, excl, 0.0), axis=1, keepdims=True)
    dest1 = (poff1 + rank1).astype(jnp.int32)
    dest2 = (poff2 + rank2).astype(jnp.int32)
    kio = lax.broadcasted_iota(jnp.int32, (N, K), 1)
    dest_ref[...] = jnp.where(kio == 0, jnp.broadcast_to(dest1, (N, K)),
                              jnp.broadcast_to(dest2, (N, K)))
    wflat_ref[...] = jnp.where(kio == 0, jnp.broadcast_to(w1n, (N, K)),
                               jnp.broadcast_to(w2n, (N, K)))

    utri = (lax.broadcasted_iota(jnp.int32, (E, E), 0)
            <= lax.broadcasted_iota(jnp.int32, (E, E), 1)).astype(jnp.float32)
    pend = jax.lax.dot_general(padded, utri, (((1,), (0,)), ((), ())),
                               precision=jax.lax.Precision.HIGHEST)
    pend_b = jnp.broadcast_to(pend, (32, E))
    it = (lax.broadcasted_iota(jnp.int32, (32, E), 0) * T).astype(jnp.float32)
    te = jnp.sum((pend_b <= it).astype(jnp.int32), axis=1, keepdims=True)
    n_tiles = jnp.sum(ptiles, axis=1, keepdims=True)
    rio = lax.broadcasted_iota(jnp.int32, (32, 1), 0)
    meta_ref[...] = jnp.where(rio < NT, te, jnp.broadcast_to(n_tiles, (32, 1)))


def _route_call(gate_logits):
    return pl.pallas_call(
        _route_body,
        grid=(1,),
        in_specs=[pl.BlockSpec((N, E), lambda i: (0, 0))],
        out_specs=[
            pl.BlockSpec((N, E), lambda i: (0, 0)),
            pl.BlockSpec((N, E), lambda i: (0, 0)),
            pl.BlockSpec((N, K), lambda i: (0, 0)),
            pl.BlockSpec((N, K), lambda i: (0, 0)),
            pl.BlockSpec((32, 1), lambda i: (0, 0)),
        ],
        out_shape=[
            jax.ShapeDtypeStruct((N, E), jnp.float32),
            jax.ShapeDtypeStruct((N, E), jnp.float32),
            jax.ShapeDtypeStruct((N, K), jnp.int32),
            jax.ShapeDtypeStruct((N, K), jnp.float32),
            jax.ShapeDtypeStruct((32, 1), jnp.int32),
        ],
    )(gate_logits)


def _sc_sort_gather(dest_flat, x):
    mesh = plsc.VectorSubcoreMesh(core_axis_name="c", subcore_axis_name="s")

    @functools.partial(
        pl.kernel, mesh=mesh,
        out_type=jax.ShapeDtypeStruct((S, D), jnp.float32),
        compiler_params=pltpu.CompilerParams(needs_layout_passes=False),
        scratch_types=[
            pltpu.VMEM((N * K,), jnp.int32),
            pltpu.VMEM((SLOTS_W,), jnp.int32),
            pltpu.VMEM((SLOTS_W // 2, D), jnp.float32),
            pltpu.SemaphoreType.DMA,
        ],
    )
    def body(dest_hbm, x_hbm, xs_hbm, dest_v, tok_v, rows_v, sem):
        wid = lax.axis_index("c") * 16 + lax.axis_index("s")
        lo = wid * SLOTS_W
        pltpu.sync_copy(dest_hbm, dest_v)
        for j in range(SLOTS_W // 16):
            tok_v[pl.ds(j * 16, 16)] = jnp.zeros((16,), jnp.int32)
        for j in range((N * K) // 16):
            d16 = dest_v[pl.ds(j * 16, 16)]
            t16 = (lax.iota(jnp.int32, 16) + j * 16) >> 1
            msk = (d16 >= lo) & (d16 < lo + SLOTS_W)
            idx16 = jnp.where(msk, d16 - lo, 0)
            plsc.store_scatter(tok_v, [idx16], t16, mask=msk)
        for hchunk in range(2):
            idx = tok_v.at[pl.ds(hchunk * (SLOTS_W // 2), SLOTS_W // 2)]
            pltpu.async_copy(x_hbm.at[idx], rows_v, sem).wait()
            pltpu.sync_copy(
                rows_v,
                xs_hbm.at[pl.ds(lo + hchunk * (SLOTS_W // 2), SLOTS_W // 2)])

    return body(dest_flat, x)


def _gmm_body(m_ref, xs_ref, W1_ref, b1_ref, W2_ref, b2_ref, ys_ref):
    @pl.when(pl.program_id(0) < m_ref[NT])
    def _():
        xb = xs_ref[...].astype(jnp.bfloat16)
        h = jnp.dot(xb, W1_ref[0], preferred_element_type=jnp.float32)
        h = jnp.maximum(h + b1_ref[0], 0.0).astype(jnp.bfloat16)
        y = jnp.dot(h, W2_ref[0], preferred_element_type=jnp.float32)
        ys_ref[...] = y + b2_ref[0]


def _gmm_call(meta_flat, xs, W1b, b1r, W2b, b2r):
    def wmap(i, m):
        e = jnp.minimum(m[i], E - 1)
        return (e, 0, 0)

    grid_spec = pltpu.PrefetchScalarGridSpec(
        num_scalar_prefetch=1,
        grid=(NT,),
        in_specs=[
            pl.BlockSpec((T, D), lambda i, m: (i, 0)),
            pl.BlockSpec((1, D, H), wmap),
            pl.BlockSpec((1, 1, H), wmap),
            pl.BlockSpec((1, H, H), wmap),
            pl.BlockSpec((1, 1, H), wmap),
        ],
        out_specs=pl.BlockSpec((T, H), lambda i, m: (i, 0)),
    )
    return pl.pallas_call(
        _gmm_body,
        grid_spec=grid_spec,
        out_shape=jax.ShapeDtypeStruct((S, H), jnp.float32),
    )(meta_flat, xs, W1b, b1r, W2b, b2r)


def _sc_pair_gather(dest_flat, ys):
    mesh = plsc.VectorSubcoreMesh(core_axis_name="c", subcore_axis_name="s")

    @functools.partial(
        pl.kernel, mesh=mesh,
        out_type=jax.ShapeDtypeStruct((N * K, H), jnp.float32),
        scratch_types=[
            pltpu.VMEM((TOKS_W,), jnp.int32),
            pltpu.VMEM((TOKS_W, H), jnp.float32),
            pltpu.SemaphoreType.DMA,
        ],
    )
    def body(dest_hbm, y_hbm, yp_hbm, idx_v, rows_v, sem):
        wid = lax.axis_index("s") * 2 + lax.axis_index("c")
        base = wid * TOKS_W
        pltpu.sync_copy(dest_hbm.at[pl.ds(base, TOKS_W)], idx_v)
        pltpu.async_copy(y_hbm.at[idx_v], rows_v, sem).wait()
        pltpu.sync_copy(rows_v, yp_hbm.at[pl.ds(base, TOKS_W)])

    return body(dest_flat, ys)


TN = 256


def _combine_body(yp_ref, w_ref, out_ref):
    yp = yp_ref[...]
    w = w_ref[...]
    kio = lax.broadcasted_iota(jnp.int32, (TN, K), 1)
    w0 = jnp.sum(jnp.where(kio == 0, w, 0.0), axis=1, keepdims=True)
    w1 = jnp.sum(jnp.where(kio == 1, w, 0.0), axis=1, keepdims=True)
    out_ref[...] = yp[:, 0, :] * w0 + yp[:, 1, :] * w1


def _combine_call(yp, wflat):
    return pl.pallas_call(
        _combine_body,
        grid=(N // TN,),
        in_specs=[
            pl.BlockSpec((TN, K, H), lambda i: (i, 0, 0)),
            pl.BlockSpec((TN, K), lambda i: (i, 0)),
        ],
        out_specs=pl.BlockSpec((TN, H), lambda i: (i, 0)),
        out_shape=jax.ShapeDtypeStruct((N, H), jnp.float32),
    )(yp, wflat)


def kernel(x, Wg, bg, W1, b1, W2, b2):
    gate_logits = x @ Wg + bg
    probs, mask, dest, wflat, meta = _route_call(gate_logits)
    dest_flat = dest.reshape(N * K)
    meta_flat = meta.reshape(32)
    W1b = W1.astype(jnp.bfloat16)
    W2b = W2.astype(jnp.bfloat16)
    b1r = b1[:, None, :]
    b2r = b2[:, None, :]
    xs = _sc_sort_gather(dest_flat, x)
    ys = _gmm_call(meta_flat, xs, W1b, b1r, W2b, b2r)
    yp = _sc_pair_gather(dest_flat, ys).reshape(N, K, H)
    out = _combine_call(yp, wflat)
    return (out, probs, mask)


# speedup vs baseline: 1.6207x; 1.6035x over previous
import functools

import jax
import jax.numpy as jnp
from jax import lax
from jax.experimental import pallas as pl
from jax.experimental.pallas import tpu as pltpu
from jax.experimental.pallas import tpu_sc as plsc

N = 2048
D = 768
H = 768
E = 8
K = 2
T = 256
NT = 24
S = NT * T
NW = 32
SLOTS_W = S // NW
TOKS_W = (N * K) // NW


def _route_body(logits_ref, probs_ref, mask_ref, dest_ref, wflat_ref,
                meta_ref):
    l = logits_ref[...]
    m = jnp.max(l, axis=-1, keepdims=True)
    ex = jnp.exp(l - m)
    probs = ex / jnp.sum(ex, axis=-1, keepdims=True)
    probs_ref[...] = probs

    iota = lax.broadcasted_iota(jnp.int32, (N, E), 1)
    m1 = jnp.max(probs, axis=-1, keepdims=True)
    i1 = jnp.min(jnp.where(probs == m1, iota, E), axis=-1, keepdims=True)
    oh1 = iota == i1
    probs2 = jnp.where(oh1, -1.0, probs)
    m2 = jnp.max(probs2, axis=-1, keepdims=True)
    i2 = jnp.min(jnp.where(probs2 == m2, iota, E), axis=-1, keepdims=True)
    oh2 = iota == i2
    denom = m1 + m2 + 1e-8
    w1n = m1 / denom
    w2n = m2 / denom
    mask_ref[...] = jnp.where(oh1, w1n, 0.0) + jnp.where(oh2, w2n, 0.0)

    ohf = (oh1 | oh2).astype(jnp.float32)
    s = ohf
    d = 1
    while d < N:
        s = jnp.concatenate([jnp.zeros((d, E), jnp.float32), s[:-d]], axis=0) + s
        d *= 2
    excl = s - ohf
    counts = s[N - 1:N, :]
    counts_i = counts.astype(jnp.int32)
    ptiles = (counts_i + (T - 1)) >> 8
    padded = (ptiles << 8).astype(jnp.float32)

    padded_b = jnp.broadcast_to(padded, (N, E))
    poff1 = jnp.sum(jnp.where(iota < i1, padded_b, 0.0), axis=1, keepdims=True)
    poff2 = jnp.sum(jnp.where(iota < i2, padded_b, 0.0), axis=1, keepdims=True)
    rank1 = jnp.sum(jnp.where(iota == i1, excl, 0.0), axis=1, keepdims=True)
    rank2 = jnp.sum(jnp.where(iota == i2, excl, 0.0), axis=1, keepdims=True)
    dest1 = (poff1 + rank1).astype(jnp.int32)
    dest2 = (poff2 + rank2).astype(jnp.int32)
    kio = lax.broadcasted_iota(jnp.int32, (N, K), 1)
    dest_ref[...] = jnp.where(kio == 0, jnp.broadcast_to(dest1, (N, K)),
                              jnp.broadcast_to(dest2, (N, K)))
    wflat_ref[...] = jnp.where(kio == 0, jnp.broadcast_to(w1n, (N, K)),
                               jnp.broadcast_to(w2n, (N, K)))

    utri = (lax.broadcasted_iota(jnp.int32, (E, E), 0)
            <= lax.broadcasted_iota(jnp.int32, (E, E), 1)).astype(jnp.float32)
    pend = jax.lax.dot_general(padded, utri, (((1,), (0,)), ((), ())),
                               precision=jax.lax.Precision.HIGHEST)
    pend_b = jnp.broadcast_to(pend, (32, E))
    it = (lax.broadcasted_iota(jnp.int32, (32, E), 0) * T).astype(jnp.float32)
    te = jnp.sum((pend_b <= it).astype(jnp.int32), axis=1, keepdims=True)
    n_tiles = jnp.sum(ptiles, axis=1, keepdims=True)
    rio = lax.broadcasted_iota(jnp.int32, (32, 1), 0)
    meta_ref[...] = jnp.where(rio < NT, te, jnp.broadcast_to(n_tiles, (32, 1)))


def _route_call(gate_logits):
    return pl.pallas_call(
        _route_body,
        grid=(1,),
        in_specs=[pl.BlockSpec((N, E), lambda i: (0, 0))],
        out_specs=[
            pl.BlockSpec((N, E), lambda i: (0, 0)),
            pl.BlockSpec((N, E), lambda i: (0, 0)),
            pl.BlockSpec((N, K), lambda i: (0, 0)),
            pl.BlockSpec((N, K), lambda i: (0, 0)),
            pl.BlockSpec((32, 1), lambda i: (0, 0)),
        ],
        out_shape=[
            jax.ShapeDtypeStruct((N, E), jnp.float32),
            jax.ShapeDtypeStruct((N, E), jnp.float32),
            jax.ShapeDtypeStruct((N, K), jnp.int32),
            jax.ShapeDtypeStruct((N, K), jnp.float32),
            jax.ShapeDtypeStruct((32, 1), jnp.int32),
        ],
    )(gate_logits)


def _sc_sort_gather(dest_flat, x):
    mesh = plsc.VectorSubcoreMesh(core_axis_name="c", subcore_axis_name="s")

    @functools.partial(
        pl.kernel, mesh=mesh,
        out_type=jax.ShapeDtypeStruct((S, D), jnp.float32),
        compiler_params=pltpu.CompilerParams(needs_layout_passes=False),
        scratch_types=[
            pltpu.VMEM((N * K,), jnp.int32),
            pltpu.VMEM((SLOTS_W,), jnp.int32),
            pltpu.VMEM((SLOTS_W // 2, D), jnp.float32),
            pltpu.SemaphoreType.DMA,
        ],
    )
    def body(dest_hbm, x_hbm, xs_hbm, dest_v, tok_v, rows_v, sem):
        wid = lax.axis_index("c") * 16 + lax.axis_index("s")
        lo = wid * SLOTS_W
        pltpu.sync_copy(dest_hbm, dest_v)
        for j in range(SLOTS_W // 16):
            tok_v[pl.ds(j * 16, 16)] = (lax.iota(jnp.int32, 16)
                                        + (lo + j * 16)) & (N - 1)
        for j in range((N * K) // 16):
            d16 = dest_v[pl.ds(j * 16, 16)]
            t16 = (lax.iota(jnp.int32, 16) + j * 16) >> 1
            msk = (d16 >= lo) & (d16 < lo + SLOTS_W)
            idx16 = jnp.where(msk, d16 - lo, 0)
            plsc.store_scatter(tok_v, [idx16], t16, mask=msk)
        for hchunk in range(2):
            idx = tok_v.at[pl.ds(hchunk * (SLOTS_W // 2), SLOTS_W // 2)]
            pltpu.async_copy(x_hbm.at[idx], rows_v, sem).wait()
            pltpu.sync_copy(
                rows_v,
                xs_hbm.at[pl.ds(lo + hchunk * (SLOTS_W // 2), SLOTS_W // 2)])

    return body(dest_flat, x)


def _gmm_body(m_ref, xs_ref, W1_ref, b1_ref, W2_ref, b2_ref, ys_ref):
    @pl.when(pl.program_id(0) < m_ref[NT])
    def _():
        xb = xs_ref[...].astype(jnp.bfloat16)
        h = jnp.dot(xb, W1_ref[0], preferred_element_type=jnp.float32)
        h = jnp.maximum(h + b1_ref[0], 0.0).astype(jnp.bfloat16)
        y = jnp.dot(h, W2_ref[0], preferred_element_type=jnp.float32)
        ys_ref[...] = y + b2_ref[0]


def _gmm_call(meta_flat, xs, W1b, b1r, W2b, b2r):
    def wmap(i, m):
        e = jnp.minimum(m[i], E - 1)
        return (e, 0, 0)

    grid_spec = pltpu.PrefetchScalarGridSpec(
        num_scalar_prefetch=1,
        grid=(NT,),
        in_specs=[
            pl.BlockSpec((T, D), lambda i, m: (i, 0)),
            pl.BlockSpec((1, D, H), wmap),
            pl.BlockSpec((1, 1, H), wmap),
            pl.BlockSpec((1, H, H), wmap),
            pl.BlockSpec((1, 1, H), wmap),
        ],
        out_specs=pl.BlockSpec((T, H), lambda i, m: (i, 0)),
    )
    return pl.pallas_call(
        _gmm_body,
        grid_spec=grid_spec,
        out_shape=jax.ShapeDtypeStruct((S, H), jnp.float32),
    )(meta_flat, xs, W1b, b1r, W2b, b2r)


def _sc_pair_gather(dest_flat, ys):
    mesh = plsc.VectorSubcoreMesh(core_axis_name="c", subcore_axis_name="s")

    @functools.partial(
        pl.kernel, mesh=mesh,
        out_type=jax.ShapeDtypeStruct((N * K, H), jnp.float32),
        scratch_types=[
            pltpu.VMEM((TOKS_W,), jnp.int32),
            pltpu.VMEM((TOKS_W, H), jnp.float32),
            pltpu.SemaphoreType.DMA,
        ],
    )
    def body(dest_hbm, y_hbm, yp_hbm, idx_v, rows_v, sem):
        wid = lax.axis_index("s") * 2 + lax.axis_index("c")
        base = wid * TOKS_W
        pltpu.sync_copy(dest_hbm.at[pl.ds(base, TOKS_W)], idx_v)
        pltpu.async_copy(y_hbm.at[idx_v], rows_v, sem).wait()
        pltpu.sync_copy(rows_v, yp_hbm.at[pl.ds(base, TOKS_W)])

    return body(dest_flat, ys)


TN = 256


def _combine_body(yp_ref, w_ref, out_ref):
    yp = yp_ref[...]
    w = w_ref[...]
    kio = lax.broadcasted_iota(jnp.int32, (TN, K), 1)
    w0 = jnp.sum(jnp.where(kio == 0, w, 0.0), axis=1, keepdims=True)
    w1 = jnp.sum(jnp.where(kio == 1, w, 0.0), axis=1, keepdims=True)
    out_ref[...] = yp[:, 0, :] * w0 + yp[:, 1, :] * w1


def _combine_call(yp, wflat):
    return pl.pallas_call(
        _combine_body,
        grid=(N // TN,),
        in_specs=[
            pl.BlockSpec((TN, K, H), lambda i: (i, 0, 0)),
            pl.BlockSpec((TN, K), lambda i: (i, 0)),
        ],
        out_specs=pl.BlockSpec((TN, H), lambda i: (i, 0)),
        out_shape=jax.ShapeDtypeStruct((N, H), jnp.float32),
    )(yp, wflat)


def kernel(x, Wg, bg, W1, b1, W2, b2):
    gate_logits = x @ Wg + bg
    probs, mask, dest, wflat, meta = _route_call(gate_logits)
    dest_flat = dest.reshape(N * K)
    meta_flat = meta.reshape(32)
    W1b = W1.astype(jnp.bfloat16)
    W2b = W2.astype(jnp.bfloat16)
    b1r = b1[:, None, :]
    b2r = b2[:, None, :]
    xs = _sc_sort_gather(dest_flat, x)
    ys = _gmm_call(meta_flat, xs, W1b, b1r, W2b, b2r)
    yp = _sc_pair_gather(dest_flat, ys).reshape(N, K, H)
    out = _combine_call(yp, wflat)
    return (out, probs, mask)


# speedup vs baseline: 2.1030x; 1.2976x over previous
import functools

import jax
import jax.numpy as jnp
from jax import lax
from jax.experimental import pallas as pl
from jax.experimental.pallas import tpu as pltpu
from jax.experimental.pallas import tpu_sc as plsc

N = 2048
D = 768
H = 768
E = 8
K = 2
T = 256
NT = 24
S = NT * T
NW = 32
SLOTS_W = S // NW
TOKS_W = (N * K) // NW


def _route_body(logits_ref, probs_ref, mask_ref, dest_ref, wflat_ref,
                meta_ref):
    l = logits_ref[...]
    m = jnp.max(l, axis=-1, keepdims=True)
    ex = jnp.exp(l - m)
    probs = ex / jnp.sum(ex, axis=-1, keepdims=True)
    probs_ref[...] = probs

    iota = lax.broadcasted_iota(jnp.int32, (N, E), 1)
    m1 = jnp.max(probs, axis=-1, keepdims=True)
    i1 = jnp.min(jnp.where(probs == m1, iota, E), axis=-1, keepdims=True)
    oh1 = iota == i1
    probs2 = jnp.where(oh1, -1.0, probs)
    m2 = jnp.max(probs2, axis=-1, keepdims=True)
    i2 = jnp.min(jnp.where(probs2 == m2, iota, E), axis=-1, keepdims=True)
    oh2 = iota == i2
    denom = m1 + m2 + 1e-8
    w1n = m1 / denom
    w2n = m2 / denom
    mask_ref[...] = jnp.where(oh1, w1n, 0.0) + jnp.where(oh2, w2n, 0.0)

    ohf = (oh1 | oh2).astype(jnp.float32)
    s = ohf
    d = 1
    while d < N:
        s = jnp.concatenate([jnp.zeros((d, E), jnp.float32), s[:-d]], axis=0) + s
        d *= 2
    excl = s - ohf
    counts = s[N - 1:N, :]
    counts_i = counts.astype(jnp.int32)
    ptiles = (counts_i + (T - 1)) >> 8
    padded = (ptiles << 8).astype(jnp.float32)

    padded_b = jnp.broadcast_to(padded, (N, E))
    poff1 = jnp.sum(jnp.where(iota < i1, padded_b, 0.0), axis=1, keepdims=True)
    poff2 = jnp.sum(jnp.where(iota < i2, padded_b, 0.0), axis=1, keepdims=True)
    rank1 = jnp.sum(jnp.where(iota == i1, excl, 0.0), axis=1, keepdims=True)
    rank2 = jnp.sum(jnp.where(iota == i2, excl, 0.0), axis=1, keepdims=True)
    dest1 = (poff1 + rank1).astype(jnp.int32)
    dest2 = (poff2 + rank2).astype(jnp.int32)
    kio = lax.broadcasted_iota(jnp.int32, (N, K), 1)
    dest_ref[...] = jnp.where(kio == 0, jnp.broadcast_to(dest1, (N, K)),
                              jnp.broadcast_to(dest2, (N, K)))
    wflat_ref[...] = jnp.where(kio == 0, jnp.broadcast_to(w1n, (N, K)),
                               jnp.broadcast_to(w2n, (N, K)))

    utri = (lax.broadcasted_iota(jnp.int32, (E, E), 0)
            <= lax.broadcasted_iota(jnp.int32, (E, E), 1)).astype(jnp.float32)
    pend = jax.lax.dot_general(padded, utri, (((1,), (0,)), ((), ())),
                               precision=jax.lax.Precision.HIGHEST)
    pend_b = jnp.broadcast_to(pend, (32, E))
    it = (lax.broadcasted_iota(jnp.int32, (32, E), 0) * T).astype(jnp.float32)
    te = jnp.sum((pend_b <= it).astype(jnp.int32), axis=1, keepdims=True)
    n_tiles = jnp.sum(ptiles, axis=1, keepdims=True)
    rio = lax.broadcasted_iota(jnp.int32, (32, 1), 0)
    meta_ref[...] = jnp.where(rio < NT, te, jnp.broadcast_to(n_tiles, (32, 1)))


def _route_call(gate_logits):
    return pl.pallas_call(
        _route_body,
        grid=(1,),
        in_specs=[pl.BlockSpec((N, E), lambda i: (0, 0))],
        out_specs=[
            pl.BlockSpec((N, E), lambda i: (0, 0)),
            pl.BlockSpec((N, E), lambda i: (0, 0)),
            pl.BlockSpec((N, K), lambda i: (0, 0)),
            pl.BlockSpec((N, K), lambda i: (0, 0)),
            pl.BlockSpec((32, 1), lambda i: (0, 0)),
        ],
        out_shape=[
            jax.ShapeDtypeStruct((N, E), jnp.float32),
            jax.ShapeDtypeStruct((N, E), jnp.float32),
            jax.ShapeDtypeStruct((N, K), jnp.int32),
            jax.ShapeDtypeStruct((N, K), jnp.float32),
            jax.ShapeDtypeStruct((32, 1), jnp.int32),
        ],
    )(gate_logits)


def _sc_sort_gather(dest_flat, x):
    mesh = plsc.VectorSubcoreMesh(core_axis_name="c", subcore_axis_name="s")

    @functools.partial(
        pl.kernel, mesh=mesh,
        out_type=jax.ShapeDtypeStruct((S, D), jnp.float32),
        compiler_params=pltpu.CompilerParams(needs_layout_passes=False),
        scratch_types=[
            pltpu.VMEM((N * K,), jnp.int32),
            pltpu.VMEM((SLOTS_W,), jnp.int32),
            pltpu.VMEM((SLOTS_W // 2, D), jnp.float32),
            pltpu.SemaphoreType.DMA,
        ],
    )
    def body(dest_hbm, x_hbm, xs_hbm, dest_v, tok_v, rows_v, sem):
        wid = lax.axis_index("c") * 16 + lax.axis_index("s")
        lo = wid * SLOTS_W
        pltpu.sync_copy(dest_hbm, dest_v)
        for j in range(SLOTS_W // 16):
            tok_v[pl.ds(j * 16, 16)] = (lax.iota(jnp.int32, 16)
                                        + (lo + j * 16)) & (N - 1)
        for j in range((N * K) // 16):
            d16 = dest_v[pl.ds(j * 16, 16)]
            t16 = (lax.iota(jnp.int32, 16) + j * 16) >> 1
            msk = (d16 >= lo) & (d16 < lo + SLOTS_W)
            idx16 = jnp.where(msk, d16 - lo, 0)
            plsc.store_scatter(tok_v, [idx16], t16, mask=msk)
        for hchunk in range(2):
            idx = tok_v.at[pl.ds(hchunk * (SLOTS_W // 2), SLOTS_W // 2)]
            pltpu.async_copy(x_hbm.at[idx], rows_v, sem).wait()
            pltpu.sync_copy(
                rows_v,
                xs_hbm.at[pl.ds(lo + hchunk * (SLOTS_W // 2), SLOTS_W // 2)])

    return body(dest_flat, x)


def _gmm_body(m_ref, xs_ref, W1_ref, b1_ref, W2_ref, b2_ref, ys_ref):
    @pl.when(pl.program_id(0) < m_ref[NT])
    def _():
        xb = xs_ref[...].astype(jnp.bfloat16)
        h = jnp.dot(xb, W1_ref[0], preferred_element_type=jnp.float32)
        h = jnp.maximum(h + b1_ref[0], 0.0).astype(jnp.bfloat16)
        y = jnp.dot(h, W2_ref[0], preferred_element_type=jnp.float32)
        ys_ref[...] = y + b2_ref[0]


def _gmm_call(meta_flat, xs, W1b, b1r, W2b, b2r):
    def wmap(i, m):
        e = jnp.minimum(m[i], E - 1)
        return (e, 0, 0)

    grid_spec = pltpu.PrefetchScalarGridSpec(
        num_scalar_prefetch=1,
        grid=(NT,),
        in_specs=[
            pl.BlockSpec((T, D), lambda i, m: (i, 0)),
            pl.BlockSpec((1, D, H), wmap),
            pl.BlockSpec((1, 1, H), wmap),
            pl.BlockSpec((1, H, H), wmap),
            pl.BlockSpec((1, 1, H), wmap),
        ],
        out_specs=pl.BlockSpec((T, H), lambda i, m: (i, 0)),
    )
    return pl.pallas_call(
        _gmm_body,
        grid_spec=grid_spec,
        out_shape=jax.ShapeDtypeStruct((S, H), jnp.float32),
    )(meta_flat, xs, W1b, b1r, W2b, b2r)


TOK_T = N // NW


def _sc_pair_gather(dest_flat, ys):
    mesh = plsc.VectorSubcoreMesh(core_axis_name="c", subcore_axis_name="s")

    @functools.partial(
        pl.kernel, mesh=mesh,
        out_type=[jax.ShapeDtypeStruct((N, H), jnp.float32),
                  jax.ShapeDtypeStruct((N, H), jnp.float32)],
        compiler_params=pltpu.CompilerParams(needs_layout_passes=False),
        scratch_types=[
            pltpu.VMEM((2 * TOK_T,), jnp.int32),
            pltpu.VMEM((TOK_T,), jnp.int32),
            pltpu.VMEM((TOK_T,), jnp.int32),
            pltpu.VMEM((TOK_T, H), jnp.float32),
            pltpu.VMEM((TOK_T, H), jnp.float32),
            pltpu.SemaphoreType.DMA,
        ],
    )
    def body(dest_hbm, y_hbm, y0_hbm, y1_hbm, dv, idx0_v, idx1_v,
             r0_v, r1_v, sem):
        wid = lax.axis_index("c") * 16 + lax.axis_index("s")
        base = wid * TOK_T
        pltpu.sync_copy(dest_hbm.at[pl.ds(base * 2, 2 * TOK_T)], dv)
        for j in range(TOK_T // 16):
            ev = lax.iota(jnp.int32, 16) * 2 + j * 32
            idx0_v[pl.ds(j * 16, 16)] = plsc.load_gather(dv, [ev])
            idx1_v[pl.ds(j * 16, 16)] = plsc.load_gather(dv, [ev + 1])
        cp0 = pltpu.async_copy(y_hbm.at[idx0_v], r0_v, sem)
        cp1 = pltpu.async_copy(y_hbm.at[idx1_v], r1_v, sem)
        cp0.wait()
        cp1.wait()
        pltpu.sync_copy(r0_v, y0_hbm.at[pl.ds(base, TOK_T)])
        pltpu.sync_copy(r1_v, y1_hbm.at[pl.ds(base, TOK_T)])

    return body(dest_flat, ys)


TN = 256


def _combine_body(y0_ref, y1_ref, w_ref, out_ref):
    w = w_ref[...]
    kio = lax.broadcasted_iota(jnp.int32, (TN, K), 1)
    w0 = jnp.sum(jnp.where(kio == 0, w, 0.0), axis=1, keepdims=True)
    w1 = jnp.sum(jnp.where(kio == 1, w, 0.0), axis=1, keepdims=True)
    out_ref[...] = y0_ref[...] * w0 + y1_ref[...] * w1


def _combine_call(y0, y1, wflat):
    return pl.pallas_call(
        _combine_body,
        grid=(N // TN,),
        in_specs=[
            pl.BlockSpec((TN, H), lambda i: (i, 0)),
            pl.BlockSpec((TN, H), lambda i: (i, 0)),
            pl.BlockSpec((TN, K), lambda i: (i, 0)),
        ],
        out_specs=pl.BlockSpec((TN, H), lambda i: (i, 0)),
        out_shape=jax.ShapeDtypeStruct((N, H), jnp.float32),
    )(y0, y1, wflat)


def kernel(x, Wg, bg, W1, b1, W2, b2):
    gate_logits = x @ Wg + bg
    probs, mask, dest, wflat, meta = _route_call(gate_logits)
    dest_flat = dest.reshape(N * K)
    meta_flat = meta.reshape(32)
    W1b = W1.astype(jnp.bfloat16)
    W2b = W2.astype(jnp.bfloat16)
    b1r = b1[:, None, :]
    b2r = b2[:, None, :]
    xs = _sc_sort_gather(dest_flat, x)
    ys = _gmm_call(meta_flat, xs, W1b, b1r, W2b, b2r)
    y0, y1 = _sc_pair_gather(dest_flat, ys)
    out = _combine_call(y0, y1, wflat)
    return (out, probs, mask)


# speedup vs baseline: 2.1624x; 1.0283x over previous
import functools

import jax
import jax.numpy as jnp
from jax import lax
from jax.experimental import pallas as pl
from jax.experimental.pallas import tpu as pltpu
from jax.experimental.pallas import tpu_sc as plsc

N = 2048
D = 768
H = 768
E = 8
K = 2
T = 256
NT = 24
S = NT * T
NW = 32
SLOTS_W = S // NW
TOKS_W = (N * K) // NW


def _route_body(logits_ref, probs_ref, mask_ref, dest_ref, wflat_ref,
                meta_ref):
    l = logits_ref[...]
    m = jnp.max(l, axis=-1, keepdims=True)
    ex = jnp.exp(l - m)
    probs = ex / jnp.sum(ex, axis=-1, keepdims=True)
    probs_ref[...] = probs

    iota = lax.broadcasted_iota(jnp.int32, (N, E), 1)
    m1 = jnp.max(probs, axis=-1, keepdims=True)
    i1 = jnp.min(jnp.where(probs == m1, iota, E), axis=-1, keepdims=True)
    oh1 = iota == i1
    probs2 = jnp.where(oh1, -1.0, probs)
    m2 = jnp.max(probs2, axis=-1, keepdims=True)
    i2 = jnp.min(jnp.where(probs2 == m2, iota, E), axis=-1, keepdims=True)
    oh2 = iota == i2
    denom = m1 + m2 + 1e-8
    w1n = m1 / denom
    w2n = m2 / denom
    mask_ref[...] = jnp.where(oh1, w1n, 0.0) + jnp.where(oh2, w2n, 0.0)

    ohf = (oh1 | oh2).astype(jnp.float32)
    s = ohf
    d = 1
    while d < N:
        s = jnp.concatenate([jnp.zeros((d, E), jnp.float32), s[:-d]], axis=0) + s
        d *= 2
    excl = s - ohf
    counts = s[N - 1:N, :]
    counts_i = counts.astype(jnp.int32)
    ptiles = (counts_i + (T - 1)) >> 8
    padded = (ptiles << 8).astype(jnp.float32)

    padded_b = jnp.broadcast_to(padded, (N, E))
    poff1 = jnp.sum(jnp.where(iota < i1, padded_b, 0.0), axis=1, keepdims=True)
    poff2 = jnp.sum(jnp.where(iota < i2, padded_b, 0.0), axis=1, keepdims=True)
    rank1 = jnp.sum(jnp.where(iota == i1, excl, 0.0), axis=1, keepdims=True)
    rank2 = jnp.sum(jnp.where(iota == i2, excl, 0.0), axis=1, keepdims=True)
    dest1 = (poff1 + rank1).astype(jnp.int32)
    dest2 = (poff2 + rank2).astype(jnp.int32)
    kio = lax.broadcasted_iota(jnp.int32, (N, K), 1)
    dest_ref[...] = jnp.where(kio == 0, jnp.broadcast_to(dest1, (N, K)),
                              jnp.broadcast_to(dest2, (N, K)))
    wflat_ref[...] = jnp.where(kio == 0, jnp.broadcast_to(w1n, (N, K)),
                               jnp.broadcast_to(w2n, (N, K)))

    utri = (lax.broadcasted_iota(jnp.int32, (E, E), 0)
            <= lax.broadcasted_iota(jnp.int32, (E, E), 1)).astype(jnp.float32)
    pend = jax.lax.dot_general(padded, utri, (((1,), (0,)), ((), ())),
                               precision=jax.lax.Precision.HIGHEST)
    pend_b = jnp.broadcast_to(pend, (32, E))
    it = (lax.broadcasted_iota(jnp.int32, (32, E), 0) * T).astype(jnp.float32)
    te = jnp.sum((pend_b <= it).astype(jnp.int32), axis=1, keepdims=True)
    n_tiles = jnp.sum(ptiles, axis=1, keepdims=True)
    rio = lax.broadcasted_iota(jnp.int32, (32, 1), 0)
    meta_ref[...] = jnp.where(rio < NT, te, jnp.broadcast_to(n_tiles, (32, 1)))


def _route_call(gate_logits):
    return pl.pallas_call(
        _route_body,
        grid=(1,),
        in_specs=[pl.BlockSpec((N, E), lambda i: (0, 0))],
        out_specs=[
            pl.BlockSpec((N, E), lambda i: (0, 0)),
            pl.BlockSpec((N, E), lambda i: (0, 0)),
            pl.BlockSpec((N, K), lambda i: (0, 0)),
            pl.BlockSpec((N, K), lambda i: (0, 0)),
            pl.BlockSpec((32, 1), lambda i: (0, 0)),
        ],
        out_shape=[
            jax.ShapeDtypeStruct((N, E), jnp.float32),
            jax.ShapeDtypeStruct((N, E), jnp.float32),
            jax.ShapeDtypeStruct((N, K), jnp.int32),
            jax.ShapeDtypeStruct((N, K), jnp.float32),
            jax.ShapeDtypeStruct((32, 1), jnp.int32),
        ],
    )(gate_logits)


def _sc_sort_gather(dest_flat, x):
    mesh = plsc.VectorSubcoreMesh(core_axis_name="c", subcore_axis_name="s")

    @functools.partial(
        pl.kernel, mesh=mesh,
        out_type=jax.ShapeDtypeStruct((S, D), jnp.float32),
        compiler_params=pltpu.CompilerParams(needs_layout_passes=False),
        scratch_types=[
            pltpu.VMEM((N * K,), jnp.int32),
            pltpu.VMEM((SLOTS_W,), jnp.int32),
            pltpu.VMEM((SLOTS_W // 2, D), jnp.float32),
            pltpu.SemaphoreType.DMA,
        ],
    )
    def body(dest_hbm, x_hbm, xs_hbm, dest_v, tok_v, rows_v, sem):
        wid = lax.axis_index("c") * 16 + lax.axis_index("s")
        lo = wid * SLOTS_W
        pltpu.sync_copy(dest_hbm, dest_v)
        for j in range(SLOTS_W // 16):
            tok_v[pl.ds(j * 16, 16)] = (lax.iota(jnp.int32, 16)
                                        + (lo + j * 16)) & (N - 1)
        for j in range((N * K) // 16):
            d16 = dest_v[pl.ds(j * 16, 16)]
            t16 = (lax.iota(jnp.int32, 16) + j * 16) >> 1
            msk = (d16 >= lo) & (d16 < lo + SLOTS_W)
            idx16 = jnp.where(msk, d16 - lo, 0)
            plsc.store_scatter(tok_v, [idx16], t16, mask=msk)
        for hchunk in range(2):
            idx = tok_v.at[pl.ds(hchunk * (SLOTS_W // 2), SLOTS_W // 2)]
            pltpu.async_copy(x_hbm.at[idx], rows_v, sem).wait()
            pltpu.sync_copy(
                rows_v,
                xs_hbm.at[pl.ds(lo + hchunk * (SLOTS_W // 2), SLOTS_W // 2)])

    return body(dest_flat, x)


def _gmm_body(m_ref, xs_ref, W1_ref, b1_ref, W2_ref, b2_ref, ys_ref):
    @pl.when(pl.program_id(0) < m_ref[NT])
    def _():
        xb = xs_ref[...].astype(jnp.bfloat16)
        h = jnp.dot(xb, W1_ref[0], preferred_element_type=jnp.float32)
        h = jnp.maximum(h + b1_ref[0], 0.0).astype(jnp.bfloat16)
        y = jnp.dot(h, W2_ref[0], preferred_element_type=jnp.float32)
        ys_ref[...] = y + b2_ref[0]


def _gmm_call(meta_flat, xs, W1b, b1r, W2b, b2r):
    def wmap(i, m):
        e = jnp.minimum(m[i], E - 1)
        return (e, 0, 0)

    def rowmap(i, m):
        return (jnp.minimum(i, m[NT] - 1), 0)

    grid_spec = pltpu.PrefetchScalarGridSpec(
        num_scalar_prefetch=1,
        grid=(NT,),
        in_specs=[
            pl.BlockSpec((T, D), rowmap),
            pl.BlockSpec((1, D, H), wmap),
            pl.BlockSpec((1, 1, H), wmap),
            pl.BlockSpec((1, H, H), wmap),
            pl.BlockSpec((1, 1, H), wmap),
        ],
        out_specs=pl.BlockSpec((T, H), rowmap),
    )
    return pl.pallas_call(
        _gmm_body,
        grid_spec=grid_spec,
        out_shape=jax.ShapeDtypeStruct((S, H), jnp.float32),
    )(meta_flat, xs, W1b, b1r, W2b, b2r)


TOK_T = N // NW


def _sc_pair_gather(dest_flat, ys):
    mesh = plsc.VectorSubcoreMesh(core_axis_name="c", subcore_axis_name="s")

    @functools.partial(
        pl.kernel, mesh=mesh,
        out_type=[jax.ShapeDtypeStruct((N, H), jnp.float32),
                  jax.ShapeDtypeStruct((N, H), jnp.float32)],
        compiler_params=pltpu.CompilerParams(needs_layout_passes=False),
        scratch_types=[
            pltpu.VMEM((2 * TOK_T,), jnp.int32),
            pltpu.VMEM((TOK_T,), jnp.int32),
            pltpu.VMEM((TOK_T,), jnp.int32),
            pltpu.VMEM((TOK_T, H), jnp.float32),
            pltpu.VMEM((TOK_T, H), jnp.float32),
            pltpu.SemaphoreType.DMA,
        ],
    )
    def body(dest_hbm, y_hbm, y0_hbm, y1_hbm, dv, idx0_v, idx1_v,
             r0_v, r1_v, sem):
        wid = lax.axis_index("c") * 16 + lax.axis_index("s")
        base = wid * TOK_T
        pltpu.sync_copy(dest_hbm.at[pl.ds(base * 2, 2 * TOK_T)], dv)
        for j in range(TOK_T // 16):
            ev = lax.iota(jnp.int32, 16) * 2 + j * 32
            idx0_v[pl.ds(j * 16, 16)] = plsc.load_gather(dv, [ev])
            idx1_v[pl.ds(j * 16, 16)] = plsc.load_gather(dv, [ev + 1])
        cp0 = pltpu.async_copy(y_hbm.at[idx0_v], r0_v, sem)
        cp1 = pltpu.async_copy(y_hbm.at[idx1_v], r1_v, sem)
        cp0.wait()
        cp1.wait()
        pltpu.sync_copy(r0_v, y0_hbm.at[pl.ds(base, TOK_T)])
        pltpu.sync_copy(r1_v, y1_hbm.at[pl.ds(base, TOK_T)])

    return body(dest_flat, ys)


TN = 256


def _combine_body(y0_ref, y1_ref, w_ref, out_ref):
    w = w_ref[...]
    kio = lax.broadcasted_iota(jnp.int32, (TN, K), 1)
    w0 = jnp.sum(jnp.where(kio == 0, w, 0.0), axis=1, keepdims=True)
    w1 = jnp.sum(jnp.where(kio == 1, w, 0.0), axis=1, keepdims=True)
    out_ref[...] = y0_ref[...] * w0 + y1_ref[...] * w1


def _combine_call(y0, y1, wflat):
    return pl.pallas_call(
        _combine_body,
        grid=(N // TN,),
        in_specs=[
            pl.BlockSpec((TN, H), lambda i: (i, 0)),
            pl.BlockSpec((TN, H), lambda i: (i, 0)),
            pl.BlockSpec((TN, K), lambda i: (i, 0)),
        ],
        out_specs=pl.BlockSpec((TN, H), lambda i: (i, 0)),
        out_shape=jax.ShapeDtypeStruct((N, H), jnp.float32),
    )(y0, y1, wflat)


def kernel(x, Wg, bg, W1, b1, W2, b2):
    gate_logits = x @ Wg + bg
    probs, mask, dest, wflat, meta = _route_call(gate_logits)
    dest_flat = dest.reshape(N * K)
    meta_flat = meta.reshape(32)
    W1b = W1.astype(jnp.bfloat16)
    W2b = W2.astype(jnp.bfloat16)
    b1r = b1[:, None, :]
    b2r = b2[:, None, :]
    xs = _sc_sort_gather(dest_flat, x)
    ys = _gmm_call(meta_flat, xs, W1b, b1r, W2b, b2r)
    y0, y1 = _sc_pair_gather(dest_flat, ys)
    out = _combine_call(y0, y1, wflat)
    return (out, probs, mask)
